# baseline (speedup 1.0000x reference)
import functools

import jax
import jax.numpy as jnp
from jax import lax
from jax.experimental import pallas as pl
from jax.experimental.pallas import tpu as pltpu
from jax.experimental.pallas import tpu_sc as plsc

FACTORS = 64
BATCH = 16384
L = 16
NC, NS = 2, 16
NW = NC * NS
RPT = BATCH // NW
CHUNK = 128
NCHUNK = RPT // CHUNK


def _fm_body(u2d, i2d, eu_t, ei_t, ub_t, ib_t, b16,
             out2d,
             uidx, iidx, eu_v, ei_v, ubv, ibv, bv, pv, ov, sem):
    cid = lax.axis_index("c")
    sid = lax.axis_index("s")
    wid = sid * NC + cid

    pltpu.sync_copy(u2d.at[pl.ds(wid * NCHUNK, NCHUNK)], uidx)
    pltpu.sync_copy(i2d.at[pl.ds(wid * NCHUNK, NCHUNK)], iidx)
    pltpu.sync_copy(b16, bv)

    copies = []
    for c in range(NCHUNK):
        copies.append(pltpu.async_copy(
            eu_t.at[uidx.at[c]], eu_v.at[pl.ds(c * CHUNK, CHUNK)], sem))
        copies.append(pltpu.async_copy(
            ei_t.at[iidx.at[c]], ei_v.at[pl.ds(c * CHUNK, CHUNK)], sem))
        copies.append(pltpu.async_copy(
            ub_t.at[uidx.at[c]], ubv.at[pl.ds(c * CHUNK, CHUNK)], sem))
        copies.append(pltpu.async_copy(
            ib_t.at[iidx.at[c]], ibv.at[pl.ds(c * CHUNK, CHUNK)], sem))
    for cp in copies:
        cp.wait()

    def row_body(j, carry):
        acc = eu_v[j, pl.ds(0, L)] * ei_v[j, pl.ds(0, L)]
        for k in range(1, FACTORS // L):
            acc = acc + eu_v[j, pl.ds(k * L, L)] * ei_v[j, pl.ds(k * L, L)]
        pv[j, :] = acc
        return carry

    lax.fori_loop(0, RPT, row_body, 0)

    iota = lax.iota(jnp.int32, L)

    def grp_body(g, carry):
        base = g * L
        rows = base + iota
        acc = bv[...] + ubv[pl.ds(base, L)] + ibv[pl.ds(base, L)]
        for l in range(L):
            col = jnp.full((L,), l, jnp.int32)
            acc = acc + plsc.load_gather(pv, [rows, col])
        ov[pl.ds(base, L)] = acc
        return carry

    lax.fori_loop(0, RPT // L, grp_body, 0)

    pltpu.sync_copy(ov, out2d.at[wid])


@jax.jit
def _fm(user2d, item2d, embed_user, embed_item, u_bias, i_bias, bias16):
    mesh = plsc.VectorSubcoreMesh(core_axis_name="c", subcore_axis_name="s")
    fn = functools.partial(
        pl.kernel,
        mesh=mesh,
        compiler_params=pltpu.CompilerParams(
            needs_layout_passes=False, use_tc_tiling_on_sc=False),
        out_type=jax.ShapeDtypeStruct((NW, RPT), jnp.float32),
        scratch_types=[
            pltpu.VMEM((NCHUNK, CHUNK), jnp.int32),
            pltpu.VMEM((NCHUNK, CHUNK), jnp.int32),
            pltpu.VMEM((RPT, FACTORS), jnp.float32),
            pltpu.VMEM((RPT, FACTORS), jnp.float32),
            pltpu.VMEM((RPT,), jnp.float32),
            pltpu.VMEM((RPT,), jnp.float32),
            pltpu.VMEM((L,), jnp.float32),
            pltpu.VMEM((RPT, L), jnp.float32),
            pltpu.VMEM((RPT,), jnp.float32),
            pltpu.SemaphoreType.DMA,
        ],
    )(_fm_body)
    return fn(user2d, item2d, embed_user, embed_item, u_bias, i_bias, bias16)


def kernel(user, item, embed_user, embed_item, u_bias, i_bias, bias_):
    user2d = user.astype(jnp.int32).reshape(NW * NCHUNK, CHUNK)
    item2d = item.astype(jnp.int32).reshape(NW * NCHUNK, CHUNK)
    bias16 = jnp.broadcast_to(bias_.reshape(1), (L,))
    out2d = _fm(user2d, item2d, embed_user, embed_item,
                u_bias.reshape(-1), i_bias.reshape(-1), bias16)
    return out2d.reshape(-1)
